# baseline (speedup 1.0000x reference)
import functools

import jax
import jax.numpy as jnp
from jax import lax
from jax.experimental import pallas as pl
from jax.experimental.pallas import tpu as pltpu
from jax.experimental.pallas import tpu_sc as plsc

K = 4
R = 2048
D = 4096
NC = 2
NS = 16
NW = NC * NS
ROWS_PER_W = R // NW


def _make_sc_upsample():
    mesh = plsc.VectorSubcoreMesh(core_axis_name="c", subcore_axis_name="s")

    @functools.partial(
        pl.kernel,
        mesh=mesh,
        out_type=jax.ShapeDtypeStruct((R, K, D), jnp.float32),
        scratch_types=[
            pltpu.SemaphoreType.DMA,
        ],
    )
    def upsample(xs_hbm, out_hbm, sem):
        wid = lax.axis_index("s") * NC + lax.axis_index("c")
        base = wid * ROWS_PER_W
        copies = []
        for r in range(K):
            copies.append(
                pltpu.async_copy(
                    xs_hbm.at[pl.ds(base, ROWS_PER_W)],
                    out_hbm.at[pl.ds(base, ROWS_PER_W), pl.ds(r, 1)],
                    sem,
                )
            )
        for cp in copies:
            cp.wait()

    return upsample


_sc_upsample = _make_sc_upsample()


def kernel(x, x_short):
    xs = x_short.reshape(R, 1, D)
    out = _sc_upsample(xs)
    return out.reshape(R * K, 4, 1024)


# speedup vs baseline: 16.0212x; 16.0212x over previous
import functools

import jax
import jax.numpy as jnp
from jax import lax
from jax.experimental import pallas as pl
from jax.experimental.pallas import tpu as pltpu
from jax.experimental.pallas import tpu_sc as plsc

K = 4
R = 2048
D = 4096
NC = 2
NS = 16
NW = NC * NS
ROWS_PER_W = R // NW
NBUF = 8


def _make_sc_upsample():
    mesh = plsc.VectorSubcoreMesh(core_axis_name="c", subcore_axis_name="s")

    G = ROWS_PER_W // NBUF

    @functools.partial(
        pl.kernel,
        mesh=mesh,
        out_type=jax.ShapeDtypeStruct((R * K, D), jnp.float32),
        scratch_types=[
            pltpu.VMEM((2 * NBUF, D), jnp.float32),
            pltpu.SemaphoreType.DMA,
            pltpu.SemaphoreType.DMA,
            pltpu.SemaphoreType.DMA,
            pltpu.SemaphoreType.DMA,
        ],
    )
    def upsample(xs_hbm, out_hbm, buf, lsem0, lsem1, ssem0, ssem1):
        wid = lax.axis_index("s") * NC + lax.axis_index("c")
        base = wid * ROWS_PER_W
        lsems = (lsem0, lsem1)
        ssems = (ssem0, ssem1)

        loads = [None] * G
        stores = [[] for _ in range(G)]

        def issue_load(g):
            par = g % 2
            return pltpu.async_copy(
                xs_hbm.at[pl.ds(base + g * NBUF, NBUF)],
                buf.at[pl.ds(par * NBUF, NBUF)],
                lsems[par],
            )

        loads[0] = issue_load(0)
        for g in range(G):
            par = g % 2
            if g + 1 < G:
                if g - 1 >= 0:
                    for st in stores[g - 1]:
                        st.wait()
                loads[g + 1] = issue_load(g + 1)
            loads[g].wait()
            row0 = base + g * NBUF
            for b in range(NBUF):
                for r in range(K):
                    stores[g].append(
                        pltpu.async_copy(
                            buf.at[pl.ds(par * NBUF + b, 1)],
                            out_hbm.at[pl.ds((row0 + b) * K + r, 1)],
                            ssems[par],
                        )
                    )
        for g in (G - 2, G - 1):
            for st in stores[g]:
                st.wait()

    return upsample


_sc_upsample = _make_sc_upsample()


def kernel(x, x_short):
    xs = x_short.reshape(R, D)
    out = _sc_upsample(xs)
    return out.reshape(R * K, 4, 1024)


# speedup vs baseline: 83.1108x; 5.1876x over previous
import functools

import jax
import jax.numpy as jnp
from jax import lax
from jax.experimental import pallas as pl
from jax.experimental.pallas import tpu as pltpu

K = 4
R = 2048
BLK = 128


def _tc_body(x_ref, o_ref):
    o_ref[...] = jnp.broadcast_to(
        x_ref[...][:, None, :, :], (BLK, K, 4, 1024)
    )


def _tc_upsample(xs):
    return pl.pallas_call(
        _tc_body,
        grid=(R // BLK,),
        in_specs=[pl.BlockSpec((BLK, 4, 1024), lambda i: (i, 0, 0))],
        out_specs=pl.BlockSpec((BLK, K, 4, 1024), lambda i: (i, 0, 0, 0)),
        out_shape=jax.ShapeDtypeStruct((R, K, 4, 1024), jnp.float32),
    )(xs)


def kernel(x, x_short):
    out = _tc_upsample(x_short)
    return out.reshape(R * K, 4, 1024)
